# baseline (speedup 1.0000x reference)
import functools

import jax
import jax.numpy as jnp
from jax import lax
from jax.experimental import pallas as pl
from jax.experimental.pallas import tpu as pltpu
from jax.experimental.pallas import tpu_sc as plsc

N = 50000
E = 300000
H = 4
D = 16
HD = H * D

CHUNK = 128
NTILE = 16
EPT = 18816
EPAD = EPT * NTILE
NCHUNK = EPT // CHUNK
NACC = 50048
RPT = NACC // NTILE

BLK = 2000
GRID = N // BLK


def _tc_pre_body(fp_ref, fa_ref, wp_ref, wa_ref, bp_ref, ba_ref, atts_ref,
                 attd_ref, s_ref, whp_ref, wha_ref, lo_ref, hi_ref,
                 asrc_ref, adst_ref):
    fp = fp_ref[...]
    fa = fa_ref[...]
    wp = jnp.dot(fp, wp_ref[...], preferred_element_type=jnp.float32) + bp_ref[...]
    wa = jnp.dot(fa, wa_ref[...], preferred_element_type=jnp.float32) + ba_ref[...]
    whp, wp2p, wp2a = wp[:, 0:64], wp[:, 64:128], wp[:, 128:192]
    wha, wa2p, wa2a = wa[:, 0:64], wa[:, 64:128], wa[:, 128:192]
    whp_ref[...] = whp
    wha_ref[...] = wha
    s_mat = s_ref[...]
    srcs = (wp2p, wp2a, wa2p, wa2a)
    dsts = (whp, wha, whp, wha)
    for r in range(4):
        lo_ref[r] = srcs[r][:, 0:32]
        hi_ref[r] = srcs[r][:, 32:64]
        a_s = jnp.dot(srcs[r] * atts_ref[r], s_mat,
                      preferred_element_type=jnp.float32)
        a_d = jnp.dot(dsts[r] * attd_ref[r], s_mat,
                      preferred_element_type=jnp.float32)
        zpad = jnp.zeros((a_s.shape[0], 12), jnp.float32)
        asrc_ref[r] = jnp.concatenate([a_s, zpad], axis=-1)
        adst_ref[r] = jnp.concatenate([a_d, zpad], axis=-1)


def _tc_pre(fp, fa, wstk_p, wstk_a, bstk_p, bstk_a, atts, attd, s_mat):
    f32 = jnp.float32
    return pl.pallas_call(
        _tc_pre_body,
        grid=(GRID,),
        in_specs=[
            pl.BlockSpec((BLK, 128), lambda i: (i, 0)),
            pl.BlockSpec((BLK, 128), lambda i: (i, 0)),
            pl.BlockSpec((128, 192), lambda i: (0, 0)),
            pl.BlockSpec((128, 192), lambda i: (0, 0)),
            pl.BlockSpec((1, 192), lambda i: (0, 0)),
            pl.BlockSpec((1, 192), lambda i: (0, 0)),
            pl.BlockSpec((4, 64), lambda i: (0, 0)),
            pl.BlockSpec((4, 64), lambda i: (0, 0)),
            pl.BlockSpec((64, 4), lambda i: (0, 0)),
        ],
        out_specs=[
            pl.BlockSpec((BLK, 64), lambda i: (i, 0)),
            pl.BlockSpec((BLK, 64), lambda i: (i, 0)),
            pl.BlockSpec((4, BLK, 32), lambda i: (0, i, 0)),
            pl.BlockSpec((4, BLK, 32), lambda i: (0, i, 0)),
            pl.BlockSpec((4, BLK, 16), lambda i: (0, i, 0)),
            pl.BlockSpec((4, BLK, 16), lambda i: (0, i, 0)),
        ],
        out_shape=[
            jax.ShapeDtypeStruct((N, 64), f32),
            jax.ShapeDtypeStruct((N, 64), f32),
            jax.ShapeDtypeStruct((4, N, 32), f32),
            jax.ShapeDtypeStruct((4, N, 32), f32),
            jax.ShapeDtypeStruct((4, N, 16), f32),
            jax.ShapeDtypeStruct((4, N, 16), f32),
        ],
    )(fp, fa, wstk_p, wstk_a, bstk_p, bstk_a, atts, attd, s_mat)


def _sc_gat(*args):
    return _build_sc_gat()(*args)


@functools.cache
def _build_sc_gat():
    return pl.kernel(
        _sc_gat_body,
        out_type=(
            jax.ShapeDtypeStruct((4, 2 * NACC, 32), jnp.float32),
            jax.ShapeDtypeStruct((2 * NACC, 32), jnp.float32),
        ),
        mesh=plsc.VectorSubcoreMesh(core_axis_name="c", subcore_axis_name="s",
                                    num_cores=2, num_subcores=16),
        compiler_params=pltpu.CompilerParams(needs_layout_passes=False,
                                             use_tc_tiling_on_sc=False),
        scratch_types=[
            pltpu.VMEM_SHARED((NACC, 32), jnp.float32),
            pltpu.VMEM((CHUNK,), jnp.int32),
            pltpu.VMEM((CHUNK,), jnp.int32),
            pltpu.VMEM((CHUNK,), jnp.int32),
            pltpu.VMEM((CHUNK,), jnp.int32),
            pltpu.VMEM((CHUNK, 16), jnp.float32),
            pltpu.VMEM((CHUNK, 16), jnp.float32),
            pltpu.VMEM((CHUNK, 32), jnp.float32),
            pltpu.VMEM((CHUNK, 32), jnp.float32),
            pltpu.VMEM((CHUNK, 32), jnp.float32),
            pltpu.VMEM((CHUNK, 32), jnp.float32),
            pltpu.SemaphoreType.DMA,
            pltpu.SemaphoreType.DMA,
            pltpu.SemaphoreType.DMA,
            pltpu.SemaphoreType.DMA,
            pltpu.SemaphoreType.DMA,
        ],
    )


def _sc_gat_body(src_hbm, dst_hbm, asrc_hbm, adst_hbm, whtab_hbm, z32_hbm,
                 ft_hbm, dn_hbm, tab_sh, sidx_v, didx_v, didx2_v, sidx2_v,
                 asrc_v, adst_v, coef_v, den_v, den2_v, rows_v,
                 sem, sem2, sem3, sem4, sem5):
    c = lax.axis_index("c")
    s = lax.axis_index("s")
    iot = lax.iota(jnp.int32, 16)
    q4 = iot >> 2
    m4 = iot & 3
    z16 = jnp.zeros((16,), jnp.float32)
    q2 = iot >> 1
    m2 = iot & 1
    ebase = s * EPT
    rb = s * RPT
    cbase = 2 * c

    def cz(t, _):
        coef_v[t >> 1, pl.ds((t & 1) * 16, 16)] = z16
        return 0

    lax.fori_loop(0, 2 * CHUNK, cz, 0)
    pltpu.sync_copy(z32_hbm, tab_sh.at[pl.ds(rb, RPT)])
    plsc.subcore_barrier()

    for r in range(4):
        def p1(j, _):
            base = ebase + j * CHUNK
            c1 = pltpu.async_copy(src_hbm.at[r, pl.ds(base, CHUNK)], sidx_v, sem)
            c2 = pltpu.async_copy(dst_hbm.at[r, pl.ds(base, CHUNK)], didx_v, sem2)
            c1.wait()
            c2.wait()
            g1 = pltpu.async_copy(asrc_hbm.at[r].at[sidx_v], asrc_v, sem)
            g2 = pltpu.async_copy(adst_hbm.at[r].at[didx_v], adst_v, sem2)
            g1.wait()
            g2.wait()

            def sub(k8, _2):
                for kk in range(4):
                    rowk = (k8 * 4 + kk) * 4 + q4
                    av = plsc.load_gather(asrc_v, [rowk, m4])
                    bv = plsc.load_gather(adst_v, [rowk, m4])
                    x = av + bv
                    x = jnp.where(x >= 0.0, x, x * 0.2)
                    plsc.store_scatter(coef_v, [rowk, m4], jnp.exp(x))
                return 0

            lax.fori_loop(0, CHUNK // 16, sub, 0)
            pltpu.sync_copy(coef_v, tab_sh.at[didx_v], add=True)
            return 0

        lax.fori_loop(0, NCHUNK, p1, 0)
        plsc.subcore_barrier()

        pltpu.sync_copy(tab_sh.at[pl.ds(rb, RPT)],
                        dn_hbm.at[pl.ds(c * NACC + rb, RPT)])
        pltpu.sync_copy(z32_hbm, tab_sh.at[pl.ds(rb, RPT)])
        plsc.subcore_barrier()

        def p2(j, _):
            base = ebase + j * CHUNK
            c1 = pltpu.async_copy(src_hbm.at[r, pl.ds(base, CHUNK)], sidx_v, sem)
            c2 = pltpu.async_copy(dst_hbm.at[r, pl.ds(base, CHUNK)], didx_v, sem2)
            c1.wait()
            c2.wait()

            def offs(u, _2):
                sl = pl.ds(u * 16, 16)
                didx2_v[sl] = didx_v[sl] + c * NACC
                sidx2_v[sl] = sidx_v[sl] + c * N
                return 0

            lax.fori_loop(0, CHUNK // 16, offs, 0)
            g1 = pltpu.async_copy(asrc_hbm.at[r].at[sidx_v], asrc_v, sem)
            g2 = pltpu.async_copy(adst_hbm.at[r].at[didx_v], adst_v, sem2)
            g3 = pltpu.async_copy(dn_hbm.at[didx2_v], den_v, sem3)
            g4 = pltpu.async_copy(whtab_hbm.at[r].at[sidx2_v], rows_v, sem4)
            g1.wait()
            g2.wait()
            g3.wait()

            def sub(k8, _2):
                for kk in range(4):
                    rowk = (k8 * 4 + kk) * 8 + q2
                    colk = cbase + m2
                    av = plsc.load_gather(asrc_v, [rowk, colk])
                    bv = plsc.load_gather(adst_v, [rowk, colk])
                    x = av + bv
                    x = jnp.where(x >= 0.0, x, x * 0.2)
                    ex = jnp.exp(x)
                    dv = plsc.load_gather(den_v, [rowk, colk])
                    plsc.store_scatter(coef_v, [rowk, colk], ex / (dv + 1e-9))
                return 0

            lax.fori_loop(0, CHUNK // 32, sub, 0)
            g4.wait()

            def scl(t8, _2):
                for tt in range(8):
                    t = t8 * 8 + tt
                    i = t >> 1
                    hh = t & 1
                    row16 = jnp.broadcast_to(i, (16,))
                    col16 = jnp.broadcast_to(cbase + hh, (16,))
                    cf = plsc.load_gather(coef_v, [row16, col16])
                    off = hh * 16
                    rows_v[i, pl.ds(off, 16)] = rows_v[i, pl.ds(off, 16)] * cf
                return 0

            lax.fori_loop(0, CHUNK // 4, scl, 0)
            pltpu.sync_copy(rows_v, tab_sh.at[didx_v], add=True)
            return 0

        lax.fori_loop(0, NCHUNK, p2, 0)
        plsc.subcore_barrier()

        pltpu.sync_copy(tab_sh.at[pl.ds(rb, RPT)],
                        ft_hbm.at[r, pl.ds(c * NACC + rb, RPT)])
        pltpu.sync_copy(z32_hbm, tab_sh.at[pl.ds(rb, RPT)])
        plsc.subcore_barrier()


def _tc_post_body(whp_ref, wha_ref, lo_ref, hi_ref, hp_ref, ha_ref):
    lo = lo_ref[...]
    hi = hi_ref[...]
    ft_p = jnp.concatenate([lo[0] + lo[2], hi[0] + hi[2]], axis=-1)
    ft_a = jnp.concatenate([lo[1] + lo[3], hi[1] + hi[3]], axis=-1)
    hp_ref[...] = jnp.maximum(whp_ref[...] + ft_p, 0.0)
    ha_ref[...] = jnp.maximum(wha_ref[...] + ft_a, 0.0)


def _tc_post(whp, wha, ftlo, fthi):
    f32 = jnp.float32
    return pl.pallas_call(
        _tc_post_body,
        grid=(GRID,),
        in_specs=[
            pl.BlockSpec((BLK, 64), lambda i: (i, 0)),
            pl.BlockSpec((BLK, 64), lambda i: (i, 0)),
            pl.BlockSpec((4, BLK, 32), lambda i: (0, i, 0)),
            pl.BlockSpec((4, BLK, 32), lambda i: (0, i, 0)),
        ],
        out_specs=[
            pl.BlockSpec((BLK, 64), lambda i: (i, 0)),
            pl.BlockSpec((BLK, 64), lambda i: (i, 0)),
        ],
        out_shape=[
            jax.ShapeDtypeStruct((N, 64), f32),
            jax.ShapeDtypeStruct((N, 64), f32),
        ],
    )(whp, wha, ftlo, fthi)


def kernel(feat_P, feat_A, edge_p2p, edge_p2a, edge_a2p, edge_a2a, W_P, b_P,
           W_A, b_A, W_p2p, b_p2p, W_p2a, b_p2a, W_a2p, b_a2p, W_a2a, b_a2a,
           attn_p2p_src, attn_p2p_dst, attn_p2a_src, attn_p2a_dst,
           attn_a2p_src, attn_a2p_dst, attn_a2a_src, attn_a2a_dst):
    f32 = jnp.float32
    i32 = jnp.int32

    wstk_p = jnp.concatenate([W_P, W_p2p, W_p2a], axis=1)
    wstk_a = jnp.concatenate([W_A, W_a2p, W_a2a], axis=1)
    bstk_p = jnp.concatenate([b_P, b_p2p, b_p2a]).reshape(1, 192)
    bstk_a = jnp.concatenate([b_A, b_a2p, b_a2a]).reshape(1, 192)
    atts = jnp.stack([attn_p2p_src.reshape(-1), attn_p2a_src.reshape(-1),
                      attn_a2p_src.reshape(-1), attn_a2a_src.reshape(-1)])
    attd = jnp.stack([attn_p2p_dst.reshape(-1), attn_p2a_dst.reshape(-1),
                      attn_a2p_dst.reshape(-1), attn_a2a_dst.reshape(-1)])
    s_mat = (jnp.arange(64)[:, None] // 16 == jnp.arange(4)[None, :]).astype(f32)

    pad = EPAD - E
    edges = (edge_p2p, edge_p2a, edge_a2p, edge_a2a)
    src_stk = jnp.stack(
        [jnp.concatenate([e[0].astype(i32), jnp.zeros((pad,), i32)])
         for e in edges])
    dst_stk = jnp.stack(
        [jnp.concatenate([e[1].astype(i32), jnp.full((pad,), N, i32)])
         for e in edges])
    z32 = jnp.zeros((RPT, 32), f32)

    whp, wha, lo, hi, asrc, adst = _tc_pre(
        feat_P, feat_A, wstk_p, wstk_a, bstk_p, bstk_a, atts, attd, s_mat)
    whtab = jnp.concatenate([lo, hi], axis=1)
    ft, _ = _sc_gat(src_stk, dst_stk, asrc, adst, whtab, z32)
    ftlo = ft[:, :N]
    fthi = ft[:, NACC:NACC + N]
    hp, ha = _tc_post(whp, wha, ftlo, fthi)
    return hp.reshape(N, H, D), ha.reshape(N, H, D)


# speedup vs baseline: 1.0366x; 1.0366x over previous
import functools

import jax
import jax.numpy as jnp
from jax import lax
from jax.experimental import pallas as pl
from jax.experimental.pallas import tpu as pltpu
from jax.experimental.pallas import tpu_sc as plsc

N = 50000
E = 300000
H = 4
D = 16
HD = H * D

CHUNK = 128
NTILE = 16
EPT = 18816
EPAD = EPT * NTILE
NCHUNK = EPT // CHUNK
NACC = 50048
RPT = NACC // NTILE

BLK = 2000
GRID = N // BLK


def _tc_pre_body(fp_ref, fa_ref, wp_ref, wa_ref, bp_ref, ba_ref, atts_ref,
                 attd_ref, s_ref, whp_ref, wha_ref, lo_ref, hi_ref,
                 asrc_ref, adst_ref):
    fp = fp_ref[...]
    fa = fa_ref[...]
    wp = jnp.dot(fp, wp_ref[...], preferred_element_type=jnp.float32) + bp_ref[...]
    wa = jnp.dot(fa, wa_ref[...], preferred_element_type=jnp.float32) + ba_ref[...]
    whp, wp2p, wp2a = wp[:, 0:64], wp[:, 64:128], wp[:, 128:192]
    wha, wa2p, wa2a = wa[:, 0:64], wa[:, 64:128], wa[:, 128:192]
    whp_ref[...] = whp
    wha_ref[...] = wha
    s_mat = s_ref[...]
    srcs = (wp2p, wp2a, wa2p, wa2a)
    dsts = (whp, wha, whp, wha)
    for r in range(4):
        lo_ref[r] = srcs[r][:, 0:32]
        hi_ref[r] = srcs[r][:, 32:64]
        a_s = jnp.dot(srcs[r] * atts_ref[r], s_mat,
                      preferred_element_type=jnp.float32)
        a_d = jnp.dot(dsts[r] * attd_ref[r], s_mat,
                      preferred_element_type=jnp.float32)
        zpad = jnp.zeros((a_s.shape[0], 12), jnp.float32)
        asrc_ref[r] = jnp.concatenate([a_s, zpad], axis=-1)
        adst_ref[r] = jnp.concatenate([a_d, zpad], axis=-1)


def _tc_pre(fp, fa, wstk_p, wstk_a, bstk_p, bstk_a, atts, attd, s_mat):
    f32 = jnp.float32
    return pl.pallas_call(
        _tc_pre_body,
        grid=(GRID,),
        in_specs=[
            pl.BlockSpec((BLK, 128), lambda i: (i, 0)),
            pl.BlockSpec((BLK, 128), lambda i: (i, 0)),
            pl.BlockSpec((128, 192), lambda i: (0, 0)),
            pl.BlockSpec((128, 192), lambda i: (0, 0)),
            pl.BlockSpec((1, 192), lambda i: (0, 0)),
            pl.BlockSpec((1, 192), lambda i: (0, 0)),
            pl.BlockSpec((4, 64), lambda i: (0, 0)),
            pl.BlockSpec((4, 64), lambda i: (0, 0)),
            pl.BlockSpec((64, 4), lambda i: (0, 0)),
        ],
        out_specs=[
            pl.BlockSpec((BLK, 64), lambda i: (i, 0)),
            pl.BlockSpec((BLK, 64), lambda i: (i, 0)),
            pl.BlockSpec((4, BLK, 32), lambda i: (0, i, 0)),
            pl.BlockSpec((4, BLK, 32), lambda i: (0, i, 0)),
            pl.BlockSpec((4, BLK, 16), lambda i: (0, i, 0)),
            pl.BlockSpec((4, BLK, 16), lambda i: (0, i, 0)),
        ],
        out_shape=[
            jax.ShapeDtypeStruct((N, 64), f32),
            jax.ShapeDtypeStruct((N, 64), f32),
            jax.ShapeDtypeStruct((4, N, 32), f32),
            jax.ShapeDtypeStruct((4, N, 32), f32),
            jax.ShapeDtypeStruct((4, N, 16), f32),
            jax.ShapeDtypeStruct((4, N, 16), f32),
        ],
    )(fp, fa, wstk_p, wstk_a, bstk_p, bstk_a, atts, attd, s_mat)


def _sc_gat(*args):
    return _build_sc_gat()(*args)


@functools.cache
def _build_sc_gat():
    return pl.kernel(
        _sc_gat_body,
        out_type=(
            jax.ShapeDtypeStruct((4, 2 * NACC, 32), jnp.float32),
            jax.ShapeDtypeStruct((2 * NACC, 32), jnp.float32),
        ),
        mesh=plsc.VectorSubcoreMesh(core_axis_name="c", subcore_axis_name="s",
                                    num_cores=2, num_subcores=16),
        compiler_params=pltpu.CompilerParams(needs_layout_passes=False,
                                             use_tc_tiling_on_sc=False),
        scratch_types=[
            pltpu.VMEM_SHARED((NACC, 32), jnp.float32),
            pltpu.VMEM((CHUNK,), jnp.int32),
            pltpu.VMEM((CHUNK,), jnp.int32),
            pltpu.VMEM((CHUNK,), jnp.int32),
            pltpu.VMEM((CHUNK,), jnp.int32),
            pltpu.VMEM((CHUNK, 16), jnp.float32),
            pltpu.VMEM((CHUNK, 16), jnp.float32),
            pltpu.VMEM((CHUNK, 32), jnp.float32),
            pltpu.VMEM((CHUNK, 32), jnp.float32),
            pltpu.VMEM((CHUNK,), jnp.int32),
            pltpu.VMEM((CHUNK,), jnp.int32),
            pltpu.VMEM((CHUNK,), jnp.int32),
            pltpu.VMEM((CHUNK,), jnp.int32),
            pltpu.VMEM((CHUNK, 16), jnp.float32),
            pltpu.VMEM((CHUNK, 16), jnp.float32),
            pltpu.VMEM((CHUNK, 32), jnp.float32),
            pltpu.VMEM((CHUNK, 32), jnp.float32),
            pltpu.VMEM((CHUNK, 32), jnp.float32),
            pltpu.SemaphoreType.DMA,
            pltpu.SemaphoreType.DMA,
            pltpu.SemaphoreType.DMA,
            pltpu.SemaphoreType.DMA,
        ],
    )


def _sc_gat_body(src_hbm, dst_hbm, asrc_hbm, adst_hbm, whtab_hbm, z32_hbm,
                 ft_hbm, dn_hbm, tab_sh,
                 sidx_v, didx_v, didx2_v, sidx2_v, asrc_v, adst_v, den_v,
                 rows_v, sidx1_v, didx1_v, didx21_v, sidx21_v, asrc1_v,
                 adst1_v, den1_v, rows1_v, coef_v,
                 isem0, isem1, gsem0, gsem1):
    c = lax.axis_index("c")
    s = lax.axis_index("s")
    iot = lax.iota(jnp.int32, 16)
    q4 = iot >> 2
    m4 = iot & 3
    z16 = jnp.zeros((16,), jnp.float32)
    q2 = iot >> 1
    m2 = iot & 1
    ebase = s * EPT
    rb = s * RPT
    cbase = 2 * c
    sidx = (sidx_v, sidx1_v)
    didx = (didx_v, didx1_v)
    didx2 = (didx2_v, didx21_v)
    sidx2 = (sidx2_v, sidx21_v)
    asrc = (asrc_v, asrc1_v)
    adst = (adst_v, adst1_v)
    den = (den_v, den1_v)
    rows = (rows_v, rows1_v)
    isem = (isem0, isem1)
    gsem = (gsem0, gsem1)

    def cz(t, _):
        coef_v[t >> 1, pl.ds((t & 1) * 16, 16)] = z16
        return 0

    lax.fori_loop(0, 2 * CHUNK, cz, 0)
    pltpu.sync_copy(z32_hbm, tab_sh.at[pl.ds(rb, RPT)])
    plsc.subcore_barrier()

    for r in range(4):
        def p1(j, _):
            base = ebase + j * CHUNK
            c1 = pltpu.async_copy(src_hbm.at[r, pl.ds(base, CHUNK)], sidx_v, isem0)
            c2 = pltpu.async_copy(dst_hbm.at[r, pl.ds(base, CHUNK)], didx_v, isem1)
            c1.wait()
            c2.wait()
            g1 = pltpu.async_copy(asrc_hbm.at[r].at[sidx_v], asrc_v, gsem0)
            g2 = pltpu.async_copy(adst_hbm.at[r].at[didx_v], adst_v, gsem1)
            g1.wait()
            g2.wait()

            def sub(k8, _2):
                for kk in range(4):
                    rowk = (k8 * 4 + kk) * 4 + q4
                    av = plsc.load_gather(asrc_v, [rowk, m4])
                    bv = plsc.load_gather(adst_v, [rowk, m4])
                    x = av + bv
                    x = jnp.where(x >= 0.0, x, x * 0.2)
                    plsc.store_scatter(coef_v, [rowk, m4], jnp.exp(x))
                return 0

            lax.fori_loop(0, CHUNK // 16, sub, 0)
            pltpu.sync_copy(coef_v, tab_sh.at[didx_v], add=True)
            return 0

        lax.fori_loop(0, NCHUNK, p1, 0)
        plsc.subcore_barrier()

        pltpu.sync_copy(tab_sh.at[pl.ds(rb, RPT)],
                        dn_hbm.at[pl.ds(c * NACC + rb, RPT)])
        pltpu.sync_copy(z32_hbm, tab_sh.at[pl.ds(rb, RPT)])
        plsc.subcore_barrier()

        def _idx_start(jj, b):
            base = ebase + jj * CHUNK
            pltpu.async_copy(src_hbm.at[r, pl.ds(base, CHUNK)], sidx[b], isem[b])
            pltpu.async_copy(dst_hbm.at[r, pl.ds(base, CHUNK)], didx[b], isem[b])

        def _idx_wait(jj, b):
            base = ebase + jj * CHUNK
            pltpu.make_async_copy(src_hbm.at[r, pl.ds(base, CHUNK)],
                                  sidx[b], isem[b]).wait()
            pltpu.make_async_copy(dst_hbm.at[r, pl.ds(base, CHUNK)],
                                  didx[b], isem[b]).wait()

        def _offs(b):
            def offs(u, _2):
                sl = pl.ds(u * 16, 16)
                didx2[b][sl] = didx[b][sl] + c * NACC
                sidx2[b][sl] = sidx[b][sl] + c * N
                return 0

            lax.fori_loop(0, CHUNK // 16, offs, 0)

        def _gather_start(b):
            pltpu.async_copy(asrc_hbm.at[r].at[sidx[b]], asrc[b], gsem[b])
            pltpu.async_copy(adst_hbm.at[r].at[didx[b]], adst[b], gsem[b])
            pltpu.async_copy(dn_hbm.at[didx2[b]], den[b], gsem[b])
            pltpu.async_copy(whtab_hbm.at[r].at[sidx2[b]], rows[b], gsem[b])

        def _gather_wait(b):
            pltpu.make_async_copy(asrc_hbm.at[r].at[sidx[b]], asrc[b],
                                  gsem[b]).wait()
            pltpu.make_async_copy(adst_hbm.at[r].at[didx[b]], adst[b],
                                  gsem[b]).wait()
            pltpu.make_async_copy(dn_hbm.at[didx2[b]], den[b], gsem[b]).wait()
            pltpu.make_async_copy(whtab_hbm.at[r].at[sidx2[b]], rows[b],
                                  gsem[b]).wait()

        def _compute_scatter(b):
            def sub(k8, _2):
                for kk in range(4):
                    rowk = (k8 * 4 + kk) * 8 + q2
                    colk = cbase + m2
                    av = plsc.load_gather(asrc[b], [rowk, colk])
                    bv = plsc.load_gather(adst[b], [rowk, colk])
                    x = av + bv
                    x = jnp.where(x >= 0.0, x, x * 0.2)
                    ex = jnp.exp(x)
                    dv = plsc.load_gather(den[b], [rowk, colk])
                    plsc.store_scatter(coef_v, [rowk, colk], ex / (dv + 1e-9))
                return 0

            lax.fori_loop(0, CHUNK // 32, sub, 0)

            def scl(t8, _2):
                for tt in range(8):
                    t = t8 * 8 + tt
                    i = t >> 1
                    hh = t & 1
                    row16 = jnp.broadcast_to(i, (16,))
                    col16 = jnp.broadcast_to(cbase + hh, (16,))
                    cf = plsc.load_gather(coef_v, [row16, col16])
                    off = hh * 16
                    rows[b][i, pl.ds(off, 16)] = rows[b][i, pl.ds(off, 16)] * cf
                return 0

            lax.fori_loop(0, CHUNK // 4, scl, 0)
            pltpu.sync_copy(rows[b], tab_sh.at[didx[b]], add=True)

        _idx_start(0, 0)
        _idx_wait(0, 0)
        _offs(0)
        _gather_start(0)
        _idx_start(1, 1)

        def p2pair(h, _):
            for b in range(2):
                jj = h * 2 + b
                _gather_wait(b)
                _compute_scatter(b)
                _idx_wait(jj + 1, 1 - b)
                _offs(1 - b)
                _gather_start(1 - b)

                @pl.when(jj <= NCHUNK - 3)
                def _pf():
                    _idx_start(jj + 2, b)

            return 0

        lax.fori_loop(0, (NCHUNK - 1) // 2, p2pair, 0)
        _gather_wait(0)
        _compute_scatter(0)
        plsc.subcore_barrier()

        pltpu.sync_copy(tab_sh.at[pl.ds(rb, RPT)],
                        ft_hbm.at[r, pl.ds(c * NACC + rb, RPT)])
        pltpu.sync_copy(z32_hbm, tab_sh.at[pl.ds(rb, RPT)])
        plsc.subcore_barrier()


def _tc_post_body(whp_ref, wha_ref, lo_ref, hi_ref, hp_ref, ha_ref):
    lo = lo_ref[...]
    hi = hi_ref[...]
    ft_p = jnp.concatenate([lo[0] + lo[2], hi[0] + hi[2]], axis=-1)
    ft_a = jnp.concatenate([lo[1] + lo[3], hi[1] + hi[3]], axis=-1)
    hp_ref[...] = jnp.maximum(whp_ref[...] + ft_p, 0.0)
    ha_ref[...] = jnp.maximum(wha_ref[...] + ft_a, 0.0)


def _tc_post(whp, wha, ftlo, fthi):
    f32 = jnp.float32
    return pl.pallas_call(
        _tc_post_body,
        grid=(GRID,),
        in_specs=[
            pl.BlockSpec((BLK, 64), lambda i: (i, 0)),
            pl.BlockSpec((BLK, 64), lambda i: (i, 0)),
            pl.BlockSpec((4, BLK, 32), lambda i: (0, i, 0)),
            pl.BlockSpec((4, BLK, 32), lambda i: (0, i, 0)),
        ],
        out_specs=[
            pl.BlockSpec((BLK, 64), lambda i: (i, 0)),
            pl.BlockSpec((BLK, 64), lambda i: (i, 0)),
        ],
        out_shape=[
            jax.ShapeDtypeStruct((N, 64), f32),
            jax.ShapeDtypeStruct((N, 64), f32),
        ],
    )(whp, wha, ftlo, fthi)


def kernel(feat_P, feat_A, edge_p2p, edge_p2a, edge_a2p, edge_a2a, W_P, b_P,
           W_A, b_A, W_p2p, b_p2p, W_p2a, b_p2a, W_a2p, b_a2p, W_a2a, b_a2a,
           attn_p2p_src, attn_p2p_dst, attn_p2a_src, attn_p2a_dst,
           attn_a2p_src, attn_a2p_dst, attn_a2a_src, attn_a2a_dst):
    f32 = jnp.float32
    i32 = jnp.int32

    wstk_p = jnp.concatenate([W_P, W_p2p, W_p2a], axis=1)
    wstk_a = jnp.concatenate([W_A, W_a2p, W_a2a], axis=1)
    bstk_p = jnp.concatenate([b_P, b_p2p, b_p2a]).reshape(1, 192)
    bstk_a = jnp.concatenate([b_A, b_a2p, b_a2a]).reshape(1, 192)
    atts = jnp.stack([attn_p2p_src.reshape(-1), attn_p2a_src.reshape(-1),
                      attn_a2p_src.reshape(-1), attn_a2a_src.reshape(-1)])
    attd = jnp.stack([attn_p2p_dst.reshape(-1), attn_p2a_dst.reshape(-1),
                      attn_a2p_dst.reshape(-1), attn_a2a_dst.reshape(-1)])
    s_mat = (jnp.arange(64)[:, None] // 16 == jnp.arange(4)[None, :]).astype(f32)

    pad = EPAD - E
    edges = (edge_p2p, edge_p2a, edge_a2p, edge_a2a)
    src_stk = jnp.stack(
        [jnp.concatenate([e[0].astype(i32), jnp.zeros((pad,), i32)])
         for e in edges])
    dst_stk = jnp.stack(
        [jnp.concatenate([e[1].astype(i32), jnp.full((pad,), N, i32)])
         for e in edges])
    z32 = jnp.zeros((RPT, 32), f32)

    whp, wha, lo, hi, asrc, adst = _tc_pre(
        feat_P, feat_A, wstk_p, wstk_a, bstk_p, bstk_a, atts, attd, s_mat)
    whtab = jnp.concatenate([lo, hi], axis=1)
    ft, _ = _sc_gat(src_stk, dst_stk, asrc, adst, whtab, z32)
    ftlo = ft[:, :N]
    fthi = ft[:, NACC:NACC + N]
    hp, ha = _tc_post(whp, wha, ftlo, fthi)
    return hp.reshape(N, H, D), ha.reshape(N, H, D)


# speedup vs baseline: 1.1591x; 1.1182x over previous
import functools

import jax
import jax.numpy as jnp
from jax import lax
from jax.experimental import pallas as pl
from jax.experimental.pallas import tpu as pltpu
from jax.experimental.pallas import tpu_sc as plsc

N = 50000
E = 300000
H = 4
D = 16
HD = H * D

CHUNK = 128
NTILE = 16
EPT = 18816
EPAD = EPT * NTILE
NCHUNK = EPT // CHUNK
NACC = 50048
RPT = NACC // NTILE

BLK = 2000
GRID = N // BLK


def _tc_pre_body(fp_ref, fa_ref, wp_ref, wa_ref, bp_ref, ba_ref, atts_ref,
                 attd_ref, s_ref, whp_ref, wha_ref, lo_ref, hi_ref,
                 asrc_ref, adst_ref):
    fp = fp_ref[...]
    fa = fa_ref[...]
    wp = jnp.dot(fp, wp_ref[...], preferred_element_type=jnp.float32) + bp_ref[...]
    wa = jnp.dot(fa, wa_ref[...], preferred_element_type=jnp.float32) + ba_ref[...]
    whp, wp2p, wp2a = wp[:, 0:64], wp[:, 64:128], wp[:, 128:192]
    wha, wa2p, wa2a = wa[:, 0:64], wa[:, 64:128], wa[:, 128:192]
    whp_ref[...] = whp
    wha_ref[...] = wha
    s_mat = s_ref[...]
    srcs = (wp2p, wp2a, wa2p, wa2a)
    dsts = (whp, wha, whp, wha)
    for r in range(4):
        lo_ref[r] = srcs[r][:, 0:32]
        hi_ref[r] = srcs[r][:, 32:64]
        a_s = jnp.dot(srcs[r] * atts_ref[r], s_mat,
                      preferred_element_type=jnp.float32)
        a_d = jnp.dot(dsts[r] * attd_ref[r], s_mat,
                      preferred_element_type=jnp.float32)
        zpad = jnp.zeros((a_s.shape[0], 12), jnp.float32)
        asrc_ref[r] = jnp.concatenate([a_s, zpad], axis=-1)
        adst_ref[r] = jnp.concatenate([a_d, zpad], axis=-1)


def _tc_pre(fp, fa, wstk_p, wstk_a, bstk_p, bstk_a, atts, attd, s_mat):
    f32 = jnp.float32
    return pl.pallas_call(
        _tc_pre_body,
        grid=(GRID,),
        in_specs=[
            pl.BlockSpec((BLK, 128), lambda i: (i, 0)),
            pl.BlockSpec((BLK, 128), lambda i: (i, 0)),
            pl.BlockSpec((128, 192), lambda i: (0, 0)),
            pl.BlockSpec((128, 192), lambda i: (0, 0)),
            pl.BlockSpec((1, 192), lambda i: (0, 0)),
            pl.BlockSpec((1, 192), lambda i: (0, 0)),
            pl.BlockSpec((4, 64), lambda i: (0, 0)),
            pl.BlockSpec((4, 64), lambda i: (0, 0)),
            pl.BlockSpec((64, 4), lambda i: (0, 0)),
        ],
        out_specs=[
            pl.BlockSpec((BLK, 64), lambda i: (i, 0)),
            pl.BlockSpec((BLK, 64), lambda i: (i, 0)),
            pl.BlockSpec((4, BLK, 32), lambda i: (0, i, 0)),
            pl.BlockSpec((4, BLK, 32), lambda i: (0, i, 0)),
            pl.BlockSpec((4, BLK, 16), lambda i: (0, i, 0)),
            pl.BlockSpec((4, BLK, 16), lambda i: (0, i, 0)),
        ],
        out_shape=[
            jax.ShapeDtypeStruct((N, 64), f32),
            jax.ShapeDtypeStruct((N, 64), f32),
            jax.ShapeDtypeStruct((4, N, 32), f32),
            jax.ShapeDtypeStruct((4, N, 32), f32),
            jax.ShapeDtypeStruct((4, N, 16), f32),
            jax.ShapeDtypeStruct((4, N, 16), f32),
        ],
    )(fp, fa, wstk_p, wstk_a, bstk_p, bstk_a, atts, attd, s_mat)


def _sc_gat(*args):
    return _build_sc_gat()(*args)


@functools.cache
def _build_sc_gat():
    return pl.kernel(
        _sc_gat_body,
        out_type=(
            jax.ShapeDtypeStruct((4, 2 * NACC, 32), jnp.float32),
            jax.ShapeDtypeStruct((2 * NACC, 32), jnp.float32),
        ),
        mesh=plsc.VectorSubcoreMesh(core_axis_name="c", subcore_axis_name="s",
                                    num_cores=2, num_subcores=16),
        compiler_params=pltpu.CompilerParams(needs_layout_passes=False,
                                             use_tc_tiling_on_sc=False),
        scratch_types=[
            pltpu.VMEM_SHARED((NACC, 32), jnp.float32),
            pltpu.VMEM((CHUNK,), jnp.int32),
            pltpu.VMEM((CHUNK,), jnp.int32),
            pltpu.VMEM((CHUNK,), jnp.int32),
            pltpu.VMEM((CHUNK,), jnp.int32),
            pltpu.VMEM((CHUNK, 16), jnp.float32),
            pltpu.VMEM((CHUNK, 16), jnp.float32),
            pltpu.VMEM((CHUNK, 32), jnp.float32),
            pltpu.VMEM((CHUNK, 32), jnp.float32),
            pltpu.VMEM((CHUNK,), jnp.int32),
            pltpu.VMEM((CHUNK,), jnp.int32),
            pltpu.VMEM((CHUNK,), jnp.int32),
            pltpu.VMEM((CHUNK,), jnp.int32),
            pltpu.VMEM((CHUNK, 16), jnp.float32),
            pltpu.VMEM((CHUNK, 16), jnp.float32),
            pltpu.VMEM((CHUNK, 32), jnp.float32),
            pltpu.VMEM((CHUNK, 32), jnp.float32),
            pltpu.VMEM((CHUNK, 32), jnp.float32),
            pltpu.SemaphoreType.DMA,
            pltpu.SemaphoreType.DMA,
            pltpu.SemaphoreType.DMA,
            pltpu.SemaphoreType.DMA,
            pltpu.SemaphoreType.DMA,
            pltpu.SemaphoreType.DMA,
        ],
    )


def _sc_gat_body(src_hbm, dst_hbm, asrc_hbm, adst_hbm, whtab_hbm, z32_hbm,
                 ft_hbm, dn_hbm, tab_sh,
                 sidx_v, didx_v, didx2_v, sidx2_v, asrc_v, adst_v, den_v,
                 rows_v, sidx1_v, didx1_v, didx21_v, sidx21_v, asrc1_v,
                 adst1_v, den1_v, rows1_v, coef_v,
                 isem0, isem1, gsem0, gsem1, ssem0, ssem1):
    c = lax.axis_index("c")
    s = lax.axis_index("s")
    iot = lax.iota(jnp.int32, 16)
    q4 = iot >> 2
    m4 = iot & 3
    z16 = jnp.zeros((16,), jnp.float32)
    q2 = iot >> 1
    m2 = iot & 1
    ebase = s * EPT
    rb = s * RPT
    cbase = 2 * c
    sidx = (sidx_v, sidx1_v)
    didx = (didx_v, didx1_v)
    didx2 = (didx2_v, didx21_v)
    sidx2 = (sidx2_v, sidx21_v)
    asrc = (asrc_v, asrc1_v)
    adst = (adst_v, adst1_v)
    den = (den_v, den1_v)
    rows = (rows_v, rows1_v)
    isem = (isem0, isem1)
    gsem = (gsem0, gsem1)
    ssem = (ssem0, ssem1)

    def cz(t, _):
        coef_v[t >> 1, pl.ds((t & 1) * 16, 16)] = z16
        return 0

    lax.fori_loop(0, 2 * CHUNK, cz, 0)
    pltpu.sync_copy(z32_hbm, tab_sh.at[pl.ds(rb, RPT)])
    plsc.subcore_barrier()

    for r in range(4):
        def _idx_start(jj, b):
            base = ebase + jj * CHUNK
            pltpu.async_copy(src_hbm.at[r, pl.ds(base, CHUNK)], sidx[b], isem[b])
            pltpu.async_copy(dst_hbm.at[r, pl.ds(base, CHUNK)], didx[b], isem[b])

        def _idx_wait(jj, b):
            base = ebase + jj * CHUNK
            pltpu.make_async_copy(src_hbm.at[r, pl.ds(base, CHUNK)],
                                  sidx[b], isem[b]).wait()
            pltpu.make_async_copy(dst_hbm.at[r, pl.ds(base, CHUNK)],
                                  didx[b], isem[b]).wait()

        def _p1_gather_start(b):
            pltpu.async_copy(asrc_hbm.at[r].at[sidx[b]], asrc[b], gsem[b])
            pltpu.async_copy(adst_hbm.at[r].at[didx[b]], adst[b], gsem[b])

        def _p1_gather_wait(b):
            pltpu.make_async_copy(asrc_hbm.at[r].at[sidx[b]], asrc[b],
                                  gsem[b]).wait()
            pltpu.make_async_copy(adst_hbm.at[r].at[didx[b]], adst[b],
                                  gsem[b]).wait()

        def _p1_compute_scatter(b):
            def sub(k8, _2):
                for kk in range(4):
                    rowk = (k8 * 4 + kk) * 8 + q2
                    colk = cbase + m2
                    av = plsc.load_gather(asrc[b], [rowk, colk])
                    bv = plsc.load_gather(adst[b], [rowk, colk])
                    x = av + bv
                    x = jnp.where(x >= 0.0, x, x * 0.2)
                    plsc.store_scatter(coef_v, [rowk, colk], jnp.exp(x))
                return 0

            lax.fori_loop(0, CHUNK // 32, sub, 0)
            pltpu.sync_copy(coef_v, tab_sh.at[didx[b]], add=True)

        _idx_start(0, 0)
        _idx_wait(0, 0)
        _p1_gather_start(0)
        _idx_start(1, 1)

        def p1pair(h, _):
            for b in range(2):
                jj = h * 2 + b
                _p1_gather_wait(b)
                _p1_compute_scatter(b)
                _idx_wait(jj + 1, 1 - b)
                _p1_gather_start(1 - b)

                @pl.when(jj <= NCHUNK - 3)
                def _pf():
                    _idx_start(jj + 2, b)

            return 0

        lax.fori_loop(0, (NCHUNK - 1) // 2, p1pair, 0)
        _p1_gather_wait(0)
        _p1_compute_scatter(0)
        plsc.subcore_barrier()

        pltpu.sync_copy(tab_sh.at[pl.ds(rb, RPT)],
                        dn_hbm.at[pl.ds(c * NACC + rb, RPT)])
        pltpu.sync_copy(z32_hbm, tab_sh.at[pl.ds(rb, RPT)])
        plsc.subcore_barrier()

        def _offs(b):
            def offs(u, _2):
                sl = pl.ds(u * 16, 16)
                didx2[b][sl] = didx[b][sl] + c * NACC
                sidx2[b][sl] = sidx[b][sl] + c * N
                return 0

            lax.fori_loop(0, CHUNK // 16, offs, 0)

        def _gather_start(b):
            pltpu.async_copy(asrc_hbm.at[r].at[sidx[b]], asrc[b], gsem[b])
            pltpu.async_copy(adst_hbm.at[r].at[didx[b]], adst[b], gsem[b])
            pltpu.async_copy(dn_hbm.at[didx2[b]], den[b], gsem[b])
            pltpu.async_copy(whtab_hbm.at[r].at[sidx2[b]], rows[b], gsem[b])

        def _gather_wait(b):
            pltpu.make_async_copy(asrc_hbm.at[r].at[sidx[b]], asrc[b],
                                  gsem[b]).wait()
            pltpu.make_async_copy(adst_hbm.at[r].at[didx[b]], adst[b],
                                  gsem[b]).wait()
            pltpu.make_async_copy(dn_hbm.at[didx2[b]], den[b], gsem[b]).wait()
            pltpu.make_async_copy(whtab_hbm.at[r].at[sidx2[b]], rows[b],
                                  gsem[b]).wait()

        def _compute_scatter(b):
            def sub(k8, _2):
                for kk in range(4):
                    rowk = (k8 * 4 + kk) * 8 + q2
                    colk = cbase + m2
                    av = plsc.load_gather(asrc[b], [rowk, colk])
                    bv = plsc.load_gather(adst[b], [rowk, colk])
                    x = av + bv
                    x = jnp.where(x >= 0.0, x, x * 0.2)
                    ex = jnp.exp(x)
                    dv = plsc.load_gather(den[b], [rowk, colk])
                    plsc.store_scatter(coef_v, [rowk, colk], ex / (dv + 1e-9))
                return 0

            lax.fori_loop(0, CHUNK // 32, sub, 0)

            def scl(t8, _2):
                for tt in range(8):
                    t = t8 * 8 + tt
                    i = t >> 1
                    hh = t & 1
                    row16 = jnp.broadcast_to(i, (16,))
                    col16 = jnp.broadcast_to(cbase + hh, (16,))
                    cf = plsc.load_gather(coef_v, [row16, col16])
                    off = hh * 16
                    rows[b][i, pl.ds(off, 16)] = rows[b][i, pl.ds(off, 16)] * cf
                return 0

            lax.fori_loop(0, CHUNK // 4, scl, 0)
            pltpu.async_copy(rows[b], tab_sh.at[didx[b]], add=True, sem=ssem[b])

        _idx_start(0, 0)
        _idx_wait(0, 0)
        _offs(0)
        _gather_start(0)
        _idx_start(1, 1)

        def _scatter_drain(b):
            pltpu.make_async_copy(rows[b], tab_sh.at[didx[b]], ssem[b]).wait()

        def p2pair(h, _):
            for b in range(2):
                jj = h * 2 + b
                _gather_wait(b)
                _compute_scatter(b)
                _idx_wait(jj + 1, 1 - b)
                _offs(1 - b)

                @pl.when(jj >= 1)
                def _dr():
                    _scatter_drain(1 - b)

                _gather_start(1 - b)

                @pl.when(jj <= NCHUNK - 3)
                def _pf():
                    _idx_start(jj + 2, b)

            return 0

        lax.fori_loop(0, (NCHUNK - 1) // 2, p2pair, 0)
        _gather_wait(0)
        _compute_scatter(0)
        _scatter_drain(1)
        _scatter_drain(0)
        plsc.subcore_barrier()

        pltpu.sync_copy(tab_sh.at[pl.ds(rb, RPT)],
                        ft_hbm.at[r, pl.ds(c * NACC + rb, RPT)])
        pltpu.sync_copy(z32_hbm, tab_sh.at[pl.ds(rb, RPT)])
        plsc.subcore_barrier()


def _tc_post_body(whp_ref, wha_ref, lo_ref, hi_ref, hp_ref, ha_ref):
    lo = lo_ref[...]
    hi = hi_ref[...]
    ft_p = jnp.concatenate([lo[0] + lo[2], hi[0] + hi[2]], axis=-1)
    ft_a = jnp.concatenate([lo[1] + lo[3], hi[1] + hi[3]], axis=-1)
    hp_ref[...] = jnp.maximum(whp_ref[...] + ft_p, 0.0)
    ha_ref[...] = jnp.maximum(wha_ref[...] + ft_a, 0.0)


def _tc_post(whp, wha, ftlo, fthi):
    f32 = jnp.float32
    return pl.pallas_call(
        _tc_post_body,
        grid=(GRID,),
        in_specs=[
            pl.BlockSpec((BLK, 64), lambda i: (i, 0)),
            pl.BlockSpec((BLK, 64), lambda i: (i, 0)),
            pl.BlockSpec((4, BLK, 32), lambda i: (0, i, 0)),
            pl.BlockSpec((4, BLK, 32), lambda i: (0, i, 0)),
        ],
        out_specs=[
            pl.BlockSpec((BLK, 64), lambda i: (i, 0)),
            pl.BlockSpec((BLK, 64), lambda i: (i, 0)),
        ],
        out_shape=[
            jax.ShapeDtypeStruct((N, 64), f32),
            jax.ShapeDtypeStruct((N, 64), f32),
        ],
    )(whp, wha, ftlo, fthi)


def kernel(feat_P, feat_A, edge_p2p, edge_p2a, edge_a2p, edge_a2a, W_P, b_P,
           W_A, b_A, W_p2p, b_p2p, W_p2a, b_p2a, W_a2p, b_a2p, W_a2a, b_a2a,
           attn_p2p_src, attn_p2p_dst, attn_p2a_src, attn_p2a_dst,
           attn_a2p_src, attn_a2p_dst, attn_a2a_src, attn_a2a_dst):
    f32 = jnp.float32
    i32 = jnp.int32

    wstk_p = jnp.concatenate([W_P, W_p2p, W_p2a], axis=1)
    wstk_a = jnp.concatenate([W_A, W_a2p, W_a2a], axis=1)
    bstk_p = jnp.concatenate([b_P, b_p2p, b_p2a]).reshape(1, 192)
    bstk_a = jnp.concatenate([b_A, b_a2p, b_a2a]).reshape(1, 192)
    atts = jnp.stack([attn_p2p_src.reshape(-1), attn_p2a_src.reshape(-1),
                      attn_a2p_src.reshape(-1), attn_a2a_src.reshape(-1)])
    attd = jnp.stack([attn_p2p_dst.reshape(-1), attn_p2a_dst.reshape(-1),
                      attn_a2p_dst.reshape(-1), attn_a2a_dst.reshape(-1)])
    s_mat = (jnp.arange(64)[:, None] // 16 == jnp.arange(4)[None, :]).astype(f32)

    pad = EPAD - E
    edges = (edge_p2p, edge_p2a, edge_a2p, edge_a2a)
    src_stk = jnp.stack(
        [jnp.concatenate([e[0].astype(i32), jnp.zeros((pad,), i32)])
         for e in edges])
    dst_stk = jnp.stack(
        [jnp.concatenate([e[1].astype(i32), jnp.full((pad,), N, i32)])
         for e in edges])
    z32 = jnp.zeros((RPT, 32), f32)

    whp, wha, lo, hi, asrc, adst = _tc_pre(
        feat_P, feat_A, wstk_p, wstk_a, bstk_p, bstk_a, atts, attd, s_mat)
    whtab = jnp.concatenate([lo, hi], axis=1)
    ft, _ = _sc_gat(src_stk, dst_stk, asrc, adst, whtab, z32)
    ftlo = ft[:, :N]
    fthi = ft[:, NACC:NACC + N]
    hp, ha = _tc_post(whp, wha, ftlo, fthi)
    return hp.reshape(N, H, D), ha.reshape(N, H, D)
